# baseline (speedup 1.0000x reference)
import jax
import jax.numpy as jnp
from jax import lax
from jax.experimental import pallas as pl
from jax.experimental.pallas import tpu as pltpu
from jax.experimental.pallas import tpu_sc as plsc

N = 8
VOCAB = 100000
DIM = 32
B = 1024
T = 20

NUM_IDX = B * T
NW = 32
CHUNK = 128
NCHUNK = NUM_IDX // CHUNK
PER_W = NCHUNK // NW
ND = N * DIM
R = 3
TBLK = 20


def _gather_body(x_hbm, w_hbm, out_hbm, idx_v, rows_v, gsem, wsem):
    cid = lax.axis_index("c")
    sid = lax.axis_index("s")
    wid = sid * 2 + cid
    base = wid * PER_W

    start = jnp.minimum((base // 8) * 8, NCHUNK - 16)
    loc = base - start
    pltpu.sync_copy(x_hbm.at[pl.ds(start, 16)], idx_v)

    def fire_gather(j):
        pltpu.async_copy(w_hbm.at[idx_v.at[loc + j]], rows_v.at[j % R],
                         gsem.at[j % R])

    def wait_gather(j):
        pltpu.make_async_copy(w_hbm.at[idx_v.at[loc]], rows_v.at[j % R],
                              gsem.at[j % R]).wait()

    def fire_write(j):
        pltpu.async_copy(rows_v.at[j % R],
                         out_hbm.at[pl.ds((base + j) * CHUNK, CHUNK)],
                         wsem.at[j % R])

    def wait_write(j):
        pltpu.make_async_copy(w_hbm.at[idx_v.at[loc]], rows_v.at[j % R],
                              wsem.at[j % R]).wait()

    for j in range(R):
        fire_gather(j)
    for j in range(PER_W):
        wait_gather(j)
        fire_write(j)
        if j + R < PER_W:
            wait_write(j)
            fire_gather(j + R)
    for j in range(PER_W - R, PER_W):
        wait_write(j)


def _relayout_body(mid_ref, out_ref):
    for t in range(TBLK):
        tr = jnp.transpose(mid_ref[t], (1, 0))
        out_ref[:, t] = jnp.reshape(tr, (N, DIM, B))


@jax.jit
def kernel(x, weight):
    x_r = jnp.reshape(jnp.transpose(x.astype(jnp.int32), (1, 0)),
                      (NCHUNK, CHUNK))
    w2d = jnp.reshape(jnp.transpose(weight, (1, 0, 2)), (VOCAB, ND))
    gather_call = pl.kernel(
        _gather_body,
        mesh=plsc.VectorSubcoreMesh(core_axis_name="c", subcore_axis_name="s"),
        out_type=jax.ShapeDtypeStruct((NUM_IDX, ND), jnp.float32),
        scratch_types=[
            pltpu.VMEM((16, CHUNK), jnp.int32),
            pltpu.VMEM((R, CHUNK, ND), jnp.float32),
            pltpu.SemaphoreType.DMA((R,)),
            pltpu.SemaphoreType.DMA((R,)),
        ],
        compiler_params=pltpu.CompilerParams(use_tc_tiling_on_sc=True),
    )
    mid = gather_call(x_r, w2d)
    mid3 = jnp.reshape(mid, (T, B, ND))

    out4 = pl.pallas_call(
        _relayout_body,
        grid=(T // TBLK,),
        in_specs=[pl.BlockSpec((TBLK, B, ND), lambda t: (t, 0, 0))],
        out_specs=pl.BlockSpec((N, TBLK, DIM, B), lambda t: (0, t, 0, 0)),
        out_shape=jax.ShapeDtypeStruct((N, T, DIM, B), jnp.float32),
    )(mid3)
    return jnp.transpose(out4, (0, 3, 1, 2))


# speedup vs baseline: 1.0363x; 1.0363x over previous
import jax
import jax.numpy as jnp
from jax import lax
from jax.experimental import pallas as pl
from jax.experimental.pallas import tpu as pltpu
from jax.experimental.pallas import tpu_sc as plsc

N = 8
VOCAB = 100000
DIM = 32
B = 1024
T = 20

NUM_IDX = B * T
NW = 32
CHUNK = 128
NCHUNK = NUM_IDX // CHUNK
PER_W = NCHUNK // NW
ND = N * DIM
R = 3
TBLK = 10


def _gather_body(x_hbm, w_hbm, out_hbm, idx_v, rows_v, gsem, wsem):
    cid = lax.axis_index("c")
    sid = lax.axis_index("s")
    wid = sid * 2 + cid
    base = wid * PER_W

    start = jnp.minimum((base // 8) * 8, NCHUNK - 16)
    loc = base - start
    pltpu.sync_copy(x_hbm.at[pl.ds(start, 16)], idx_v)

    def fire_gather(j):
        pltpu.async_copy(w_hbm.at[idx_v.at[loc + j]], rows_v.at[j % R],
                         gsem.at[j % R])

    def wait_gather(j):
        pltpu.make_async_copy(w_hbm.at[idx_v.at[loc]], rows_v.at[j % R],
                              gsem.at[j % R]).wait()

    def fire_write(j):
        pltpu.async_copy(rows_v.at[j % R],
                         out_hbm.at[pl.ds((base + j) * CHUNK, CHUNK)],
                         wsem.at[j % R])

    def wait_write(j):
        pltpu.make_async_copy(w_hbm.at[idx_v.at[loc]], rows_v.at[j % R],
                              wsem.at[j % R]).wait()

    for j in range(R):
        fire_gather(j)
    for j in range(PER_W):
        wait_gather(j)
        fire_write(j)
        if j + R < PER_W:
            wait_write(j)
            fire_gather(j + R)
    for j in range(PER_W - R, PER_W):
        wait_write(j)


def _relayout_body(mid_ref, out_ref):
    for t in range(TBLK):
        tr = jnp.transpose(mid_ref[t], (1, 0))
        out_ref[:, t] = jnp.reshape(tr, (N, DIM, B))


@jax.jit
def kernel(x, weight):
    x_r = jnp.reshape(jnp.transpose(x.astype(jnp.int32), (1, 0)),
                      (NCHUNK, CHUNK))
    w2d = jnp.reshape(jnp.transpose(weight, (1, 0, 2)), (VOCAB, ND))
    gather_call = pl.kernel(
        _gather_body,
        mesh=plsc.VectorSubcoreMesh(core_axis_name="c", subcore_axis_name="s"),
        out_type=jax.ShapeDtypeStruct((NUM_IDX, ND), jnp.float32),
        scratch_types=[
            pltpu.VMEM((16, CHUNK), jnp.int32),
            pltpu.VMEM((R, CHUNK, ND), jnp.float32),
            pltpu.SemaphoreType.DMA((R,)),
            pltpu.SemaphoreType.DMA((R,)),
        ],
        compiler_params=pltpu.CompilerParams(use_tc_tiling_on_sc=True),
    )
    mid = gather_call(x_r, w2d)
    mid3 = jnp.reshape(mid, (T, B, ND))

    out4 = pl.pallas_call(
        _relayout_body,
        grid=(T // TBLK,),
        in_specs=[pl.BlockSpec((TBLK, B, ND), lambda t: (t, 0, 0))],
        out_specs=pl.BlockSpec((N, TBLK, DIM, B), lambda t: (0, t, 0, 0)),
        out_shape=jax.ShapeDtypeStruct((N, T, DIM, B), jnp.float32),
    )(mid3)
    return jnp.transpose(out4, (0, 3, 1, 2))
